# baseline (speedup 1.0000x reference)
import jax
import jax.numpy as jnp
from jax import lax
from jax.experimental import pallas as pl
from jax.experimental.pallas import tpu as pltpu

K_NN = 15
QB = 128
NGRP = 1792
NCK = 7
DEPTH = 4
_HI = jax.lax.Precision.HIGHEST

_TAU = 0
_VST = 1
_IE = 2
_M = 4
_CLT = 8
_IEK = 12
_R = 128


def _prep_body(xr_ref, p30_ref, dt_ref, qa0_ref, dat_ref):
    xr = xr_ref[...]
    xf = xr - jnp.mean(xr, axis=1, keepdims=True)
    xf = xf / jnp.sqrt(jnp.sum(xf * xf, axis=1, keepdims=True))
    proj = jnp.dot(xf, p30_ref[...],
                   preferred_element_type=jnp.float32)
    nq = jnp.sum(proj * proj, axis=1, keepdims=True)
    ci = lax.broadcasted_iota(jnp.int32, proj.shape, 1)
    d = proj.shape[1] - 2
    qa0_ref[...] = jnp.where(ci == d, 1.0,
                             jnp.where(ci == d + 1, nq, -2.0 * proj))
    dt = dt_ref[...]
    nd = jnp.sum(dt * dt, axis=0, keepdims=True)
    ri = lax.broadcasted_iota(jnp.int32, dt.shape, 0)
    dat_ref[...] = jnp.where(ri == d, nd, jnp.where(ri == d + 1, 0.0, dt))


def _main_body(qa0_ref, dat_ref, lab_ref, out_ref, z_scr, m_scr, res_scr):
    p = pl.program_id(1)
    inf = jnp.float32(jnp.inf)
    qb, np_ = z_scr.shape
    ck = np_ // NCK
    d = qa0_ref.shape[1] - 2

    @pl.when(p == 0)
    def _compute_cd():
        s2 = jnp.dot(qa0_ref[:, :d], dat_ref[:d, :],
                     preferred_element_type=jnp.float32)
        z_scr[...] = s2
        m_scr[...] = jnp.full(m_scr.shape, inf, jnp.float32)

    def _cd(sl):
        nq = qa0_ref[:, d + 1:d + 2]
        a = dat_ref[d:d + 1, sl] + nq
        return jnp.sqrt(jnp.maximum(a + z_scr[:, sl], 1e-12))

    for i in range(NCK):
        @pl.when(p == 1 + i)
        def _group_min(i=i):
            gm = m_scr[:, _R:_R + NGRP]
            for j in range(ck // NGRP):
                sl = slice(i * ck + j * NGRP, i * ck + (j + 1) * NGRP)
                gm = jnp.minimum(gm, _cd(sl))
            m_scr[:, _R:_R + NGRP] = gm

    @pl.when(p == NCK + 1)
    def _threshold():
        r = m_scr[:, _R:_R + NGRP]
        mprev = jnp.full((qb, 1), -inf, jnp.float32)
        for _ in range(K_NN):
            mprev = jnp.min(jnp.where(r > mprev, r, inf), axis=1,
                            keepdims=True)
        m_scr[:, _TAU:_TAU + 1] = mprev
        m_scr[:, _M:_M + DEPTH] = jnp.full((qb, DEPTH), -inf, jnp.float32)

    for i in range(NCK):
        @pl.when(p == NCK + 2 + i)
        def _refine(i=i):
            tau = m_scr[:, _TAU:_TAU + 1]
            zc = _cd(slice(i * ck, (i + 1) * ck))
            v = jnp.where(zc <= tau, zc, -inf)
            loc = []
            bound = inf
            for _ in range(DEPTH):
                mk = jnp.max(jnp.where(v < bound, v, -inf), axis=1,
                             keepdims=True)
                loc.append(mk)
                bound = mk
            u = jnp.concatenate([m_scr[:, _M:_M + DEPTH]] + loc, axis=1)
            merged = []
            bound = inf
            for _ in range(DEPTH):
                mk = jnp.max(jnp.where(u < bound, u, -inf), axis=1,
                             keepdims=True)
                merged.append(mk)
                bound = mk
            m_scr[:, _M:_M + DEPTH] = jnp.concatenate(merged, axis=1)

    @pl.when(p == 2 * NCK + 2)
    def _init_counts():
        m_scr[:, _CLT:_CLT + DEPTH] = jnp.zeros((qb, DEPTH), jnp.float32)
        m_scr[:, _IEK:_IEK + DEPTH] = jnp.full((qb, DEPTH), inf, jnp.float32)

    for i in range(NCK):
        @pl.when(p == 2 * NCK + 3 + i)
        def _tiebreak(i=i):
            zc = _cd(slice(i * ck, (i + 1) * ck))
            idx = (lax.broadcasted_iota(jnp.int32, zc.shape, 1)
                   ).astype(jnp.float32) + float(i * ck)
            for k in range(DEPTH):
                mk = m_scr[:, _M + k:_M + k + 1]
                m_scr[:, _CLT + k:_CLT + k + 1] += jnp.sum(
                    jnp.where(zc < mk, 1.0, 0.0), axis=1, keepdims=True)
                m_scr[:, _IEK + k:_IEK + k + 1] = jnp.minimum(
                    m_scr[:, _IEK + k:_IEK + k + 1],
                    jnp.min(jnp.where(zc == mk, idx, inf), axis=1,
                            keepdims=True))

    @pl.when(p == 3 * NCK + 3)
    def _pick():
        vst = m_scr[:, _M + DEPTH - 1:_M + DEPTH]
        ie = m_scr[:, _IEK + DEPTH - 1:_IEK + DEPTH]
        for k in range(DEPTH - 2, -1, -1):
            ok = m_scr[:, _CLT + k:_CLT + k + 1] < float(K_NN)
            vst = jnp.where(ok, m_scr[:, _M + k:_M + k + 1], vst)
            ie = jnp.where(ok, m_scr[:, _IEK + k:_IEK + k + 1], ie)
        m_scr[:, _VST:_VST + 1] = vst
        m_scr[:, _IE:_IE + 1] = ie
        res_scr[...] = jnp.zeros(res_scr.shape, jnp.float32)

    for i in range(NCK):
        @pl.when(p == 3 * NCK + 4 + i)
        def _weighted_labels(i=i):
            zc = _cd(slice(i * ck, (i + 1) * ck))
            idx = (lax.broadcasted_iota(jnp.int32, zc.shape, 1)
                   ).astype(jnp.float32) + float(i * ck)
            vst = m_scr[:, _VST:_VST + 1]
            sel = (zc < vst) | ((zc == vst) & (idx <= m_scr[:, _IE:_IE + 1]))
            w = jnp.where(sel, jnp.exp(-zc), 0.0)
            res_scr[...] += jnp.dot(w.astype(jnp.bfloat16),
                                    lab_ref[i * ck:(i + 1) * ck, :],
                                    preferred_element_type=jnp.float32)

    @pl.when(p == 4 * NCK + 4)
    def _finalize():
        out_ref[...] = jnp.log(res_scr[:, :out_ref.shape[1]])


def kernel(x, projector, data, labels):
    B = x.shape[0]
    n_db, d_proj = data.shape
    n_cls = labels.shape[1]
    xr = x.reshape(B, -1)
    da = d_proj + 2
    np_ = pl.cdiv(n_db, NGRP * NCK) * NGRP * NCK
    pad_rows = np_ - n_db
    lab_cols = pl.cdiv(n_cls, 8) * 8

    p30 = jnp.pad(projector[:, :d_proj], ((0, 0), (0, 2)))
    dt = jnp.concatenate(
        [data, jnp.full((pad_rows, d_proj), 1e3, jnp.float32)], axis=0)
    dt_t = jnp.pad(dt.T, ((0, 2), (0, 0)))
    lab_p = jnp.pad(labels, ((0, pad_rows), (0, lab_cols - n_cls))
                    ).astype(jnp.bfloat16)

    qa0, dat = pl.pallas_call(
        _prep_body,
        out_shape=(
            jax.ShapeDtypeStruct((B, da), jnp.float32),
            jax.ShapeDtypeStruct((da, np_), jnp.float32),
        ),
    )(xr, p30, dt_t)

    nqb = B // QB
    out = pl.pallas_call(
        _main_body,
        grid=(nqb, 4 * NCK + 5),
        in_specs=[
            pl.BlockSpec((QB, da), lambda qb, p: (qb, 0)),
            pl.BlockSpec((da, np_), lambda qb, p: (0, 0)),
            pl.BlockSpec((np_, lab_cols), lambda qb, p: (0, 0)),
        ],
        out_specs=pl.BlockSpec((QB, n_cls), lambda qb, p: (qb, 0)),
        out_shape=jax.ShapeDtypeStruct((B, n_cls), jnp.float32),
        scratch_shapes=[
            pltpu.VMEM((QB, np_), jnp.float32),
            pltpu.VMEM((QB, _R + NGRP), jnp.float32),
            pltpu.VMEM((QB, 16), jnp.float32),
        ],
    )(qa0, dat, lab_p)
    return out


# speedup vs baseline: 1.4757x; 1.4757x over previous
import jax
import jax.numpy as jnp
from jax import lax
from jax.experimental import pallas as pl
from jax.experimental.pallas import tpu as pltpu

K_NN = 15
QB = 64
NGRP = 1792
NCK = 4
DEPTH = 4
_HI = jax.lax.Precision.HIGHEST

_TAU = 0
_VST = 1
_IE = 2
_M = 4
_CLT = 8
_IEK = 12
_R = 128


def _prep_body(xr_ref, p30_ref, dt_ref, qa0_ref, dat_ref):
    xr = xr_ref[...]
    xf = xr - jnp.mean(xr, axis=1, keepdims=True)
    xf = xf / jnp.sqrt(jnp.sum(xf * xf, axis=1, keepdims=True))
    proj = jnp.dot(xf, p30_ref[...],
                   preferred_element_type=jnp.float32)
    nq = jnp.sum(proj * proj, axis=1, keepdims=True)
    ci = lax.broadcasted_iota(jnp.int32, proj.shape, 1)
    d = proj.shape[1] - 2
    qa0_ref[...] = jnp.where(ci == d, 1.0,
                             jnp.where(ci == d + 1, nq, -2.0 * proj))
    dt = dt_ref[...]
    nd = jnp.sum(dt * dt, axis=0, keepdims=True)
    ri = lax.broadcasted_iota(jnp.int32, dt.shape, 0)
    dat_ref[...] = jnp.where(ri == d, nd, jnp.where(ri == d + 1, 0.0, dt))


def _main_body(qa0_ref, dat_ref, lab_ref, out_ref, z_scr, a_scr, m_scr,
               res_scr):
    p = pl.program_id(1)
    inf = jnp.float32(jnp.inf)
    qb, np_ = z_scr.shape
    ck = np_ // NCK
    d = qa0_ref.shape[1] - 2

    @pl.when(p == 0)
    def _compute_s2():
        z_scr[...] = jnp.dot(qa0_ref[:, :d], dat_ref[:d, :],
                             preferred_element_type=jnp.float32)
        m_scr[...] = jnp.full(m_scr.shape, inf, jnp.float32)

    @pl.when(p == 1)
    def _compute_a():
        a_scr[...] = dat_ref[d:d + 1, :] + qa0_ref[:, d + 1:d + 2]

    def _cd(sl):
        return jnp.sqrt(jnp.maximum(a_scr[:, sl] + z_scr[:, sl], 1e-12))

    for i in range(NCK):
        @pl.when(p == 2 + i)
        def _group_min(i=i):
            gm = m_scr[:, _R:_R + NGRP]
            for j in range(ck // NGRP):
                sl = slice(i * ck + j * NGRP, i * ck + (j + 1) * NGRP)
                gm = jnp.minimum(gm, _cd(sl))
            m_scr[:, _R:_R + NGRP] = gm

    @pl.when(p == NCK + 2)
    def _threshold():
        r = m_scr[:, _R:_R + NGRP]
        mprev = jnp.full((qb, 1), -inf, jnp.float32)
        for _ in range(K_NN):
            mprev = jnp.min(jnp.where(r > mprev, r, inf), axis=1,
                            keepdims=True)
        m_scr[:, _TAU:_TAU + 1] = mprev
        m_scr[:, _M:_M + DEPTH] = jnp.full((qb, DEPTH), -inf, jnp.float32)

    for i in range(NCK):
        @pl.when(p == NCK + 3 + i)
        def _refine(i=i):
            tau = m_scr[:, _TAU:_TAU + 1]
            zc = _cd(slice(i * ck, (i + 1) * ck))
            v = jnp.where(zc <= tau, zc, -inf)
            loc = []
            bound = inf
            for _ in range(DEPTH):
                mk = jnp.max(jnp.where(v < bound, v, -inf), axis=1,
                             keepdims=True)
                loc.append(mk)
                bound = mk
            u = jnp.concatenate([m_scr[:, _M:_M + DEPTH]] + loc, axis=1)
            merged = []
            bound = inf
            for _ in range(DEPTH):
                mk = jnp.max(jnp.where(u < bound, u, -inf), axis=1,
                             keepdims=True)
                merged.append(mk)
                bound = mk
            m_scr[:, _M:_M + DEPTH] = jnp.concatenate(merged, axis=1)

    @pl.when(p == 2 * NCK + 3)
    def _init_counts():
        m_scr[:, _CLT:_CLT + DEPTH] = jnp.zeros((qb, DEPTH), jnp.float32)
        m_scr[:, _IEK:_IEK + DEPTH] = jnp.full((qb, DEPTH), inf, jnp.float32)

    for i in range(NCK):
        @pl.when(p == 2 * NCK + 4 + i)
        def _tiebreak(i=i):
            zc = _cd(slice(i * ck, (i + 1) * ck))
            idx = (lax.broadcasted_iota(jnp.int32, zc.shape, 1)
                   ).astype(jnp.float32) + float(i * ck)
            for k in range(DEPTH):
                mk = m_scr[:, _M + k:_M + k + 1]
                m_scr[:, _CLT + k:_CLT + k + 1] += jnp.sum(
                    jnp.where(zc < mk, 1.0, 0.0), axis=1, keepdims=True)
                m_scr[:, _IEK + k:_IEK + k + 1] = jnp.minimum(
                    m_scr[:, _IEK + k:_IEK + k + 1],
                    jnp.min(jnp.where(zc == mk, idx, inf), axis=1,
                            keepdims=True))

    @pl.when(p == 3 * NCK + 4)
    def _pick():
        vst = m_scr[:, _M + DEPTH - 1:_M + DEPTH]
        ie = m_scr[:, _IEK + DEPTH - 1:_IEK + DEPTH]
        for k in range(DEPTH - 2, -1, -1):
            ok = m_scr[:, _CLT + k:_CLT + k + 1] < float(K_NN)
            vst = jnp.where(ok, m_scr[:, _M + k:_M + k + 1], vst)
            ie = jnp.where(ok, m_scr[:, _IEK + k:_IEK + k + 1], ie)
        m_scr[:, _VST:_VST + 1] = vst
        m_scr[:, _IE:_IE + 1] = ie
        res_scr[...] = jnp.zeros(res_scr.shape, jnp.float32)

    for i in range(NCK):
        @pl.when(p == 3 * NCK + 5 + i)
        def _weighted_labels(i=i):
            zc = _cd(slice(i * ck, (i + 1) * ck))
            idx = (lax.broadcasted_iota(jnp.int32, zc.shape, 1)
                   ).astype(jnp.float32) + float(i * ck)
            vst = m_scr[:, _VST:_VST + 1]
            sel = (zc < vst) | ((zc == vst) & (idx <= m_scr[:, _IE:_IE + 1]))
            w = jnp.where(sel, jnp.exp(-zc), 0.0)
            res_scr[...] += jnp.dot(w.astype(jnp.bfloat16),
                                    lab_ref[i * ck:(i + 1) * ck, :],
                                    preferred_element_type=jnp.float32)

    @pl.when(p == 4 * NCK + 5)
    def _finalize():
        out_ref[...] = jnp.log(res_scr[:, :out_ref.shape[1]])


def kernel(x, projector, data, labels):
    B = x.shape[0]
    n_db, d_proj = data.shape
    n_cls = labels.shape[1]
    xr = x.reshape(B, -1)
    da = d_proj + 2
    np_ = pl.cdiv(n_db, NGRP * NCK) * NGRP * NCK
    pad_rows = np_ - n_db
    lab_cols = pl.cdiv(n_cls, 8) * 8

    p30 = jnp.pad(projector[:, :d_proj], ((0, 0), (0, 2)))
    dt = jnp.concatenate(
        [data, jnp.full((pad_rows, d_proj), 1e3, jnp.float32)], axis=0)
    dt_t = jnp.pad(dt.T, ((0, 2), (0, 0)))
    lab_p = jnp.pad(labels, ((0, pad_rows), (0, lab_cols - n_cls))
                    ).astype(jnp.bfloat16)

    qa0, dat = pl.pallas_call(
        _prep_body,
        out_shape=(
            jax.ShapeDtypeStruct((B, da), jnp.float32),
            jax.ShapeDtypeStruct((da, np_), jnp.float32),
        ),
    )(xr, p30, dt_t)

    nqb = B // QB
    out = pl.pallas_call(
        _main_body,
        grid=(nqb, 4 * NCK + 6),
        in_specs=[
            pl.BlockSpec((QB, da), lambda qb, p: (qb, 0)),
            pl.BlockSpec((da, np_), lambda qb, p: (0, 0)),
            pl.BlockSpec((np_, lab_cols), lambda qb, p: (0, 0)),
        ],
        out_specs=pl.BlockSpec((QB, n_cls), lambda qb, p: (qb, 0)),
        out_shape=jax.ShapeDtypeStruct((B, n_cls), jnp.float32),
        scratch_shapes=[
            pltpu.VMEM((QB, np_), jnp.float32),
            pltpu.VMEM((QB, np_), jnp.float32),
            pltpu.VMEM((QB, _R + NGRP), jnp.float32),
            pltpu.VMEM((QB, 16), jnp.float32),
        ],
    )(qa0, dat, lab_p)
    return out


# speedup vs baseline: 40.0522x; 27.1406x over previous
import jax
import jax.numpy as jnp
from jax import lax
from jax.experimental import pallas as pl
from jax.experimental.pallas import tpu as pltpu

K_NN = 15
QB = 64
NGRP = 1792
NCK = 4
DEPTH = 4


def _prep_body(xr_ref, p30_ref, dt_ref, qa0_ref, dat_ref):
    xr = xr_ref[...]
    xf = xr - jnp.mean(xr, axis=1, keepdims=True)
    xf = xf / jnp.sqrt(jnp.sum(xf * xf, axis=1, keepdims=True))
    proj = jnp.dot(xf, p30_ref[...],
                   preferred_element_type=jnp.float32)
    nq = jnp.sum(proj * proj, axis=1, keepdims=True)
    ci = lax.broadcasted_iota(jnp.int32, proj.shape, 1)
    d = proj.shape[1] - 2
    qa0_ref[...] = jnp.where(ci == d, 1.0,
                             jnp.where(ci == d + 1, nq, -2.0 * proj))
    dt = dt_ref[...]
    nd = jnp.sum(dt * dt, axis=0, keepdims=True)
    ri = lax.broadcasted_iota(jnp.int32, dt.shape, 0)
    dat_ref[...] = jnp.where(ri == d, nd, jnp.where(ri == d + 1, 0.0, dt))


def _main_body(qa0_ref, dat_ref, lab_ref, out_ref, z_scr, a_scr, r_scr):
    inf = jnp.float32(jnp.inf)
    qb, np_ = z_scr.shape
    ck = np_ // NCK
    d = qa0_ref.shape[1] - 2

    z_scr[...] = jnp.dot(qa0_ref[:, :d], dat_ref[:d, :],
                         preferred_element_type=jnp.float32)
    a_scr[...] = dat_ref[d:d + 1, :] + qa0_ref[:, d + 1:d + 2]

    gm = jnp.full((qb, NGRP), inf, jnp.float32)
    for i in range(NCK * (ck // NGRP)):
        sl = slice(i * NGRP, (i + 1) * NGRP)
        cd = jnp.sqrt(jnp.maximum(a_scr[:, sl] + z_scr[:, sl], 1e-12))
        z_scr[:, sl] = cd
        gm = jnp.minimum(gm, cd)
    r_scr[...] = gm

    mprev = jnp.full((qb, 1), -inf, jnp.float32)
    for _ in range(K_NN):
        mprev = jnp.min(jnp.where(r_scr[...] > mprev, r_scr[...], inf),
                        axis=1, keepdims=True)
    tau = mprev

    tops = [jnp.full((qb, 1), -inf, jnp.float32)] * DEPTH
    for i in range(NCK):
        zc = z_scr[:, i * ck:(i + 1) * ck]
        v = jnp.where(zc <= tau, zc, -inf)
        loc = []
        bound = inf
        for _ in range(DEPTH):
            mk = jnp.max(jnp.where(v < bound, v, -inf), axis=1,
                         keepdims=True)
            loc.append(mk)
            bound = mk
        u = jnp.concatenate(tops + loc, axis=1)
        tops = []
        bound = inf
        for _ in range(DEPTH):
            mk = jnp.max(jnp.where(u < bound, u, -inf), axis=1,
                         keepdims=True)
            tops.append(mk)
            bound = mk

    clt = [jnp.zeros((qb, 1), jnp.float32) for _ in range(DEPTH)]
    for i in range(NCK):
        zc = z_scr[:, i * ck:(i + 1) * ck]
        for k in range(DEPTH):
            clt[k] = clt[k] + jnp.sum(jnp.where(zc < tops[k], 1.0, 0.0),
                                      axis=1, keepdims=True)

    vst = tops[DEPTH - 1]
    for k in range(DEPTH - 2, -1, -1):
        vst = jnp.where(clt[k] < float(K_NN), tops[k], vst)

    ie = jnp.full((qb, 1), inf, jnp.float32)
    for i in range(NCK):
        zc = z_scr[:, i * ck:(i + 1) * ck]
        idx = (lax.broadcasted_iota(jnp.int32, zc.shape, 1)
               ).astype(jnp.float32) + float(i * ck)
        ie = jnp.minimum(ie, jnp.min(jnp.where(zc == vst, idx, inf),
                                     axis=1, keepdims=True))

    res = jnp.zeros((qb, lab_ref.shape[1]), jnp.float32)
    for i in range(NCK):
        zc = z_scr[:, i * ck:(i + 1) * ck]
        idx = (lax.broadcasted_iota(jnp.int32, zc.shape, 1)
               ).astype(jnp.float32) + float(i * ck)
        sel = (zc < vst) | ((zc == vst) & (idx <= ie))
        w = jnp.where(sel, jnp.exp(-zc), 0.0)
        res = res + jnp.dot(w.astype(jnp.bfloat16),
                            lab_ref[i * ck:(i + 1) * ck, :],
                            preferred_element_type=jnp.float32)
    out_ref[...] = jnp.log(res[:, :out_ref.shape[1]])


def kernel(x, projector, data, labels):
    B = x.shape[0]
    n_db, d_proj = data.shape
    n_cls = labels.shape[1]
    xr = x.reshape(B, -1)
    da = d_proj + 2
    np_ = pl.cdiv(n_db, NGRP * NCK) * NGRP * NCK
    pad_rows = np_ - n_db
    lab_cols = pl.cdiv(n_cls, 8) * 8

    p30 = jnp.pad(projector[:, :d_proj], ((0, 0), (0, 2)))
    dt = jnp.concatenate(
        [data, jnp.full((pad_rows, d_proj), 1e3, jnp.float32)], axis=0)
    dt_t = jnp.pad(dt.T, ((0, 2), (0, 0)))
    lab_p = jnp.pad(labels, ((0, pad_rows), (0, lab_cols - n_cls))
                    ).astype(jnp.bfloat16)

    qa0, dat = pl.pallas_call(
        _prep_body,
        out_shape=(
            jax.ShapeDtypeStruct((B, da), jnp.float32),
            jax.ShapeDtypeStruct((da, np_), jnp.float32),
        ),
    )(xr, p30, dt_t)

    nqb = B // QB
    out = pl.pallas_call(
        _main_body,
        grid=(nqb,),
        in_specs=[
            pl.BlockSpec((QB, da), lambda qb: (qb, 0)),
            pl.BlockSpec((da, np_), lambda qb: (0, 0)),
            pl.BlockSpec((np_, lab_cols), lambda qb: (0, 0)),
        ],
        out_specs=pl.BlockSpec((QB, n_cls), lambda qb: (qb, 0)),
        out_shape=jax.ShapeDtypeStruct((B, n_cls), jnp.float32),
        scratch_shapes=[
            pltpu.VMEM((QB, np_), jnp.float32),
            pltpu.VMEM((QB, np_), jnp.float32),
            pltpu.VMEM((QB, NGRP), jnp.float32),
        ],
    )(qa0, dat, lab_p)
    return out
